# Initial kernel scaffold; baseline (speedup 1.0000x reference)
import jax
import jax.numpy as jnp
from jax.experimental import pallas as pl


def kernel(x, w_gate, W1, b1, W2, b2):
    raise NotImplementedError("write your pallas kernel here")


# speedup vs baseline: 5.7222x; 5.7222x over previous
import functools

import jax
import jax.numpy as jnp
from jax.experimental import pallas as pl
from jax.experimental.pallas import tpu as pltpu

_DIM = 96
_E = 8


def _moe_body(x_ref, wg_ref, W1_ref, b1_ref, W2_ref, b2_ref,
              y_ref, loss_ref, imp_ref, load_ref, *, nb):
    b = pl.program_id(0)
    xb = x_ref[0]
    xg = jnp.mean(xb, axis=0, keepdims=True)
    logits = jnp.dot(xg, wg_ref[...],
                     preferred_element_type=jnp.float32)
    logits = jnp.clip(logits, -50.0, 50.0)
    iota = jax.lax.broadcasted_iota(jnp.int32, (1, _E), 1)
    v0 = jnp.max(logits)
    e0 = jnp.min(jnp.where(logits == v0, iota, _E))
    masked = jnp.where(iota == e0, -jnp.inf, logits)
    v1 = jnp.max(masked)
    e1 = jnp.min(jnp.where(masked == v1, iota, _E))
    t = jnp.exp(v1 - v0)
    g0 = 1.0 / (1.0 + t)
    g1 = t / (1.0 + t)

    gates_row = (jnp.where(iota == e0, g0, 0.0)
                 + jnp.where(iota == e1, g1, 0.0))

    @pl.when(b == 0)
    def _init():
        imp_ref[...] = jnp.zeros_like(imp_ref)
        load_ref[...] = jnp.zeros_like(load_ref)

    imp_ref[...] += gates_row
    load_ref[...] += (gates_row > 0.0).astype(jnp.float32)

    def expert(e):
        h = jnp.dot(xb, W1_ref[e], preferred_element_type=jnp.float32)
        h = h + b1_ref[e][None, :]
        h = 0.5 * h * (1.0 + jax.lax.erf(h * 0.7071067811865476))
        o = jnp.dot(h, W2_ref[e], preferred_element_type=jnp.float32)
        return o + b2_ref[e][None, :]

    y_ref[0] = g0 * expert(e0) + g1 * expert(e1)

    @pl.when(b == nb - 1)
    def _loss():
        eps = 1e-10
        n = float(_E)

        def cv_sq(v):
            m = jnp.sum(v) / n
            var = jnp.sum((v - m) ** 2) / (n - 1.0)
            return var / (m * m + eps)

        loss = cv_sq(imp_ref[...]) + cv_sq(load_ref[...])
        loss_ref[...] = jnp.clip(loss, 0.0, 1000.0) * jnp.ones((1, 1), jnp.float32)


def kernel(x, w_gate, W1, b1, W2, b2):
    B, H, W, C = x.shape
    T = H * W
    E = w_gate.shape[1]
    x_flat = x.reshape(B, T, C)

    y_flat, loss = pl.pallas_call(
        functools.partial(_moe_body, nb=B),
        grid=(B,),
        in_specs=[
            pl.BlockSpec((1, T, C), lambda b: (b, 0, 0)),
            pl.BlockSpec((C, E), lambda b: (0, 0)),
            pl.BlockSpec(W1.shape, lambda b: (0, 0, 0)),
            pl.BlockSpec(b1.shape, lambda b: (0, 0)),
            pl.BlockSpec(W2.shape, lambda b: (0, 0, 0)),
            pl.BlockSpec(b2.shape, lambda b: (0, 0)),
        ],
        out_specs=[
            pl.BlockSpec((1, T, C), lambda b: (b, 0, 0)),
            pl.BlockSpec((1, 1), lambda b: (0, 0)),
        ],
        out_shape=[
            jax.ShapeDtypeStruct((B, T, C), jnp.float32),
            jax.ShapeDtypeStruct((1, 1), jnp.float32),
        ],
        scratch_shapes=[
            pltpu.VMEM((1, E), jnp.float32),
            pltpu.VMEM((1, E), jnp.float32),
        ],
    )(x_flat, w_gate, W1, b1, W2, b2)

    return y_flat.reshape(B, H, W, C), loss[0, 0]


# speedup vs baseline: 6.2021x; 1.0839x over previous
import functools

import jax
import jax.numpy as jnp
from jax.experimental import pallas as pl
from jax.experimental.pallas import tpu as pltpu

_E = 8


def _moe_body(x_ref, wg_ref, W1_ref, b1_ref, W2_ref, b2_ref,
              y_ref, gates_ref):
    xb = x_ref[0]
    xg = jnp.mean(xb, axis=0, keepdims=True)
    logits = jnp.dot(xg, wg_ref[...],
                     preferred_element_type=jnp.float32)
    logits = jnp.clip(logits, -50.0, 50.0)
    iota = jax.lax.broadcasted_iota(jnp.int32, (1, _E), 1)
    v0 = jnp.max(logits)
    e0 = jnp.min(jnp.where(logits == v0, iota, _E))
    masked = jnp.where(iota == e0, -jnp.inf, logits)
    v1 = jnp.max(masked)
    e1 = jnp.min(jnp.where(masked == v1, iota, _E))
    t = jnp.exp(v1 - v0)
    g0 = 1.0 / (1.0 + t)
    g1 = t / (1.0 + t)

    gates_ref[...] = (jnp.where(iota == e0, g0, 0.0)
                      + jnp.where(iota == e1, g1, 0.0))[None]

    def expert(e):
        h = jnp.dot(xb, W1_ref[e], preferred_element_type=jnp.float32)
        h = h + b1_ref[e][None, :]
        h = 0.5 * h * (1.0 + jax.lax.erf(h * 0.7071067811865476))
        o = jnp.dot(h, W2_ref[e], preferred_element_type=jnp.float32)
        return o + b2_ref[e][None, :]

    y_ref[0] = g0 * expert(e0) + g1 * expert(e1)


def _loss_body(gates_ref, loss_ref):
    g = gates_ref[...][:, 0, :]
    n = float(_E)
    eps = 1e-10

    def cv_sq(v):
        m = jnp.sum(v) / n
        var = jnp.sum((v - m) ** 2) / (n - 1.0)
        return var / (m * m + eps)

    imp = jnp.sum(g, axis=0, keepdims=True)
    load = jnp.sum((g > 0.0).astype(jnp.float32), axis=0, keepdims=True)
    loss = cv_sq(imp) + cv_sq(load)
    loss_ref[...] = jnp.clip(loss, 0.0, 1000.0) * jnp.ones((1, 1), jnp.float32)


def kernel(x, w_gate, W1, b1, W2, b2):
    B, H, W, C = x.shape
    T = H * W
    E = w_gate.shape[1]
    x_flat = x.reshape(B, T, C)

    y_flat, gates = pl.pallas_call(
        _moe_body,
        grid=(B,),
        in_specs=[
            pl.BlockSpec((1, T, C), lambda b: (b, 0, 0)),
            pl.BlockSpec((C, E), lambda b: (0, 0)),
            pl.BlockSpec(W1.shape, lambda b: (0, 0, 0)),
            pl.BlockSpec(b1.shape, lambda b: (0, 0)),
            pl.BlockSpec(W2.shape, lambda b: (0, 0, 0)),
            pl.BlockSpec(b2.shape, lambda b: (0, 0)),
        ],
        out_specs=[
            pl.BlockSpec((1, T, C), lambda b: (b, 0, 0)),
            pl.BlockSpec((1, 1, E), lambda b: (b, 0, 0)),
        ],
        out_shape=[
            jax.ShapeDtypeStruct((B, T, C), jnp.float32),
            jax.ShapeDtypeStruct((B, 1, E), jnp.float32),
        ],
        compiler_params=pltpu.CompilerParams(
            dimension_semantics=("parallel",),
        ),
    )(x_flat, w_gate, W1, b1, W2, b2)

    loss = pl.pallas_call(
        _loss_body,
        out_shape=jax.ShapeDtypeStruct((1, 1), jnp.float32),
    )(gates)

    return y_flat.reshape(B, H, W, C), loss[0, 0]
